# baseline (speedup 1.0000x reference)
import numpy as np

import jax
import jax.numpy as jnp
from jax.experimental import pallas as pl
from jax.experimental.pallas import tpu as pltpu

B, C, H, W = 4, 96, 224, 224
NPLANE = B * C
PIX = H * W
SUB, LANE = 392, 128
PB = 8
GRID = NPLANE // PB
T = 5
KSIZE = 7
SIGMA = 1.5


def _threefry_np(k1, k2, x0, x1):
    np.seterr(over="ignore")
    ks0 = np.uint32(k1)
    ks1 = np.uint32(k2)
    ks2 = ks0 ^ ks1 ^ np.uint32(0x1BD11BDA)

    def rot(x, r):
        return (x << np.uint32(r)) | (x >> np.uint32(32 - r))

    def rounds(a, b, rots):
        for r in rots:
            a = a + b
            b = rot(b, r)
            b = a ^ b
        return a, b

    R0 = (13, 15, 26, 6)
    R1 = (17, 29, 16, 24)
    x0 = x0 + ks0
    x1 = x1 + ks1
    x0, x1 = rounds(x0, x1, R0)
    x0 += ks1
    x1 += ks2 + np.uint32(1)
    x0, x1 = rounds(x0, x1, R1)
    x0 += ks2
    x1 += ks0 + np.uint32(2)
    x0, x1 = rounds(x0, x1, R0)
    x0 += ks0
    x1 += ks1 + np.uint32(3)
    x0, x1 = rounds(x0, x1, R1)
    x0 += ks1
    x1 += ks2 + np.uint32(4)
    x0, x1 = rounds(x0, x1, R0)
    x0 += ks2
    x1 += ks0 + np.uint32(5)
    return x0, x1


_KEYS = tuple(
    tuple(int(v) for v in _threefry_np(np.uint32(0), np.uint32(42),
                                       np.uint32(0), np.uint32(t)))
    for t in range(T)
)


def _rotl(x, r):
    return (x << jnp.uint32(r)) | (x >> jnp.uint32(32 - r))


def _tf_rounds(x0, x1, rots):
    for r in rots:
        x0 = x0 + x1
        x1 = _rotl(x1, r)
        x1 = x0 ^ x1
    return x0, x1


def _threefry_bits(k1, k2, cnt):
    R0 = (13, 15, 26, 6)
    R1 = (17, 29, 16, 24)
    ks0 = np.uint32(k1)
    ks1 = np.uint32(k2)
    ks2 = np.uint32(ks0 ^ ks1 ^ np.uint32(0x1BD11BDA))
    x0 = jnp.full_like(cnt, ks0)
    x1 = cnt + ks1
    x0, x1 = _tf_rounds(x0, x1, R0)
    x0 = x0 + ks1
    x1 = x1 + np.uint32(ks2 + np.uint32(1))
    x0, x1 = _tf_rounds(x0, x1, R1)
    x0 = x0 + ks2
    x1 = x1 + np.uint32(ks0 + np.uint32(2))
    x0, x1 = _tf_rounds(x0, x1, R0)
    x0 = x0 + ks0
    x1 = x1 + np.uint32(ks1 + np.uint32(3))
    x0, x1 = _tf_rounds(x0, x1, R1)
    x0 = x0 + ks1
    x1 = x1 + np.uint32(ks2 + np.uint32(4))
    x0, x1 = _tf_rounds(x0, x1, R0)
    x0 = x0 + ks2
    x1 = x1 + np.uint32(ks0 + np.uint32(5))
    return x0 ^ x1


def _sample_body(xs_ref, maxv_ref, am_ref):
    g = pl.program_id(0)
    xs = xs_ref[...]
    loc = (jax.lax.broadcasted_iota(jnp.int32, (PB, SUB, LANE), 1) * LANE
           + jax.lax.broadcasted_iota(jnp.int32, (PB, SUB, LANE), 2))
    plane = g * PB + jax.lax.broadcasted_iota(jnp.int32, (PB, SUB, LANE), 0)
    cnt = (plane * PIX + loc).astype(jnp.uint32)
    for t in range(T):
        k1, k2 = _KEYS[t]
        bits = _threefry_bits(k1, k2, cnt)
        fb = (bits >> jnp.uint32(9)) | jnp.uint32(0x3F800000)
        u = jax.lax.bitcast_convert_type(fb, jnp.float32) - jnp.float32(1.0)
        crnd = xs * u
        m = jnp.max(jnp.max(crnd, axis=2), axis=1)
        ismax = crnd == m[:, None, None]
        idx = jnp.where(ismax, loc, PIX)
        am = jnp.min(jnp.min(idx, axis=2), axis=1)
        maxv_ref[0, t, :] = m
        am_ref[0, t, :] = am


def _att_body(wr_ref, wc_ref, out_ref):
    out_ref[...] = wr_ref[...][:, :, None] * wc_ref[...][:, None, :]


def _profiles(pos, amp, k1d):
    rp = jnp.arange(H, dtype=jnp.int32)
    p = pos[..., None]
    out = jnp.zeros((B, C, H), jnp.float32)
    pad = KSIZE // 2
    images = [
        (p + pad, jnp.full_like(p, True, dtype=bool)),
        (pad - p, (p >= 1) & (p <= pad)),
        (2 * (H - 1) + pad - p, (p >= H - 1 - pad) & (p <= H - 2)),
    ]
    for q, valid in images:
        d = q - rp
        for tap in range(KSIZE):
            out = out + jnp.where(valid & (d == tap), k1d[tap],
                                  jnp.float32(0.0))
    return out * amp[..., None]


def kernel(_x_):
    xs = jax.nn.sigmoid(_x_)
    xs3 = xs.reshape(NPLANE, SUB, LANE)

    maxv, am = pl.pallas_call(
        _sample_body,
        grid=(GRID,),
        in_specs=[pl.BlockSpec((PB, SUB, LANE), lambda g: (g, 0, 0))],
        out_specs=[
            pl.BlockSpec((1, T, PB), lambda g: (g, 0, 0)),
            pl.BlockSpec((1, T, PB), lambda g: (g, 0, 0)),
        ],
        out_shape=[
            jax.ShapeDtypeStruct((GRID, T, PB), jnp.float32),
            jax.ShapeDtypeStruct((GRID, T, PB), jnp.int32),
        ],
        compiler_params=pltpu.CompilerParams(
            dimension_semantics=("parallel",)),
    )(xs3)

    maxv = maxv.transpose(1, 0, 2).reshape(T, B, C)
    am = am.transpose(1, 0, 2).reshape(T, B, C)

    i = (am // W).astype(jnp.float32) / H
    j = (am % W).astype(jnp.float32) / W
    y = jnp.stack([i, j], axis=3)
    vis_t = maxv >= 0.1
    y = y * vis_t[..., None].astype(jnp.float32)

    m_joints = jnp.mean(y, axis=0)
    v_joints = jnp.var(y, axis=0, ddof=1)
    vis = jnp.mean(vis_t.astype(jnp.float32), axis=0) > 0.5
    visf = vis[..., None].astype(jnp.float32)
    m_joints = m_joints * visf
    v_joints = v_joints * visf

    rows = jnp.clip(jnp.round(m_joints[..., 0] * H).astype(jnp.int32), 0, H - 1)
    cols = jnp.clip(jnp.round(m_joints[..., 1] * W).astype(jnp.int32), 0, W - 1)

    half = (KSIZE - 1) * 0.5
    xs1 = jnp.linspace(-half, half, KSIZE)
    pdf = jnp.exp(-0.5 * (xs1 / SIGMA) ** 2)
    k1d = (pdf / jnp.sum(pdf)).astype(jnp.float32)

    amp = vis.astype(jnp.float32)
    wr = _profiles(rows, amp, k1d)
    wc = _profiles(cols, amp, k1d)
    gmax = jnp.max(jnp.max(wr, axis=2) * jnp.max(wc, axis=2))
    wrn = (wr / gmax).reshape(NPLANE, H)
    wcf = wc.reshape(NPLANE, W)

    att = pl.pallas_call(
        _att_body,
        grid=(GRID,),
        in_specs=[
            pl.BlockSpec((PB, H), lambda g: (g, 0)),
            pl.BlockSpec((PB, W), lambda g: (g, 0)),
        ],
        out_specs=pl.BlockSpec((PB, H, W), lambda g: (g, 0, 0)),
        out_shape=jax.ShapeDtypeStruct((NPLANE, H, W), jnp.float32),
        compiler_params=pltpu.CompilerParams(
            dimension_semantics=("arbitrary",)),
    )(wrn, wcf)

    return att.reshape(B, C, H, W), m_joints, v_joints, vis


# speedup vs baseline: 1.1163x; 1.1163x over previous
import numpy as np

import jax
import jax.numpy as jnp
from jax.experimental import pallas as pl
from jax.experimental.pallas import tpu as pltpu

B, C, H, W = 4, 96, 224, 224
NPLANE = B * C
PIX = H * W
SUB, LANE = 392, 128
PB = 16
GRID = NPLANE // PB
T = 5
KSIZE = 7
SIGMA = 1.5


def _threefry_np(k1, k2, x0, x1):
    np.seterr(over="ignore")
    ks0 = np.uint32(k1)
    ks1 = np.uint32(k2)
    ks2 = ks0 ^ ks1 ^ np.uint32(0x1BD11BDA)

    def rot(x, r):
        return (x << np.uint32(r)) | (x >> np.uint32(32 - r))

    def rounds(a, b, rots):
        for r in rots:
            a = a + b
            b = rot(b, r)
            b = a ^ b
        return a, b

    R0 = (13, 15, 26, 6)
    R1 = (17, 29, 16, 24)
    x0 = x0 + ks0
    x1 = x1 + ks1
    x0, x1 = rounds(x0, x1, R0)
    x0 += ks1
    x1 += ks2 + np.uint32(1)
    x0, x1 = rounds(x0, x1, R1)
    x0 += ks2
    x1 += ks0 + np.uint32(2)
    x0, x1 = rounds(x0, x1, R0)
    x0 += ks0
    x1 += ks1 + np.uint32(3)
    x0, x1 = rounds(x0, x1, R1)
    x0 += ks1
    x1 += ks2 + np.uint32(4)
    x0, x1 = rounds(x0, x1, R0)
    x0 += ks2
    x1 += ks0 + np.uint32(5)
    return x0, x1


_KEYS = tuple(
    tuple(int(v) for v in _threefry_np(np.uint32(0), np.uint32(42),
                                       np.uint32(0), np.uint32(t)))
    for t in range(T)
)


def _rotl(x, r):
    return (x << jnp.uint32(r)) | (x >> jnp.uint32(32 - r))


def _tf_rounds(x0, x1, rots):
    for r in rots:
        x0 = x0 + x1
        x1 = _rotl(x1, r)
        x1 = x0 ^ x1
    return x0, x1


def _threefry_bits(k1, k2, cnt):
    R0 = (13, 15, 26, 6)
    R1 = (17, 29, 16, 24)
    ks0 = np.uint32(k1)
    ks1 = np.uint32(k2)
    ks2 = np.uint32(ks0 ^ ks1 ^ np.uint32(0x1BD11BDA))
    x0 = jnp.full_like(cnt, ks0)
    x1 = cnt + ks1
    x0, x1 = _tf_rounds(x0, x1, R0)
    x0 = x0 + ks1
    x1 = x1 + np.uint32(ks2 + np.uint32(1))
    x0, x1 = _tf_rounds(x0, x1, R1)
    x0 = x0 + ks2
    x1 = x1 + np.uint32(ks0 + np.uint32(2))
    x0, x1 = _tf_rounds(x0, x1, R0)
    x0 = x0 + ks0
    x1 = x1 + np.uint32(ks1 + np.uint32(3))
    x0, x1 = _tf_rounds(x0, x1, R1)
    x0 = x0 + ks1
    x1 = x1 + np.uint32(ks2 + np.uint32(4))
    x0, x1 = _tf_rounds(x0, x1, R0)
    x0 = x0 + ks2
    x1 = x1 + np.uint32(ks0 + np.uint32(5))
    return x0 ^ x1


def _sample_body(xs_ref, maxv_ref, am_ref):
    g = pl.program_id(0)
    xs = xs_ref[...]
    loc = (jax.lax.broadcasted_iota(jnp.int32, (PB, SUB, LANE), 1) * LANE
           + jax.lax.broadcasted_iota(jnp.int32, (PB, SUB, LANE), 2))
    plane = g * PB + jax.lax.broadcasted_iota(jnp.int32, (PB, SUB, LANE), 0)
    cnt = (plane * PIX + loc).astype(jnp.uint32)
    for t in range(T):
        k1, k2 = _KEYS[t]
        bits = _threefry_bits(k1, k2, cnt)
        fb = (bits >> jnp.uint32(9)) | jnp.uint32(0x3F800000)
        u = jax.lax.bitcast_convert_type(fb, jnp.float32) - jnp.float32(1.0)
        crnd = xs * u
        m = jnp.max(jnp.max(crnd, axis=2), axis=1)
        ismax = crnd == m[:, None, None]
        idx = jnp.where(ismax, loc, PIX)
        am = jnp.min(jnp.min(idx, axis=2), axis=1)
        maxv_ref[0, t, :] = m
        am_ref[0, t, :] = am


def _att_body(wr_ref, wc_ref, out_ref):
    out_ref[...] = wr_ref[...][:, :, None] * wc_ref[...][:, None, :]


def _profiles(pos, amp, k1d):
    rp = jnp.arange(H, dtype=jnp.int32)
    p = pos[..., None]
    out = jnp.zeros((B, C, H), jnp.float32)
    pad = KSIZE // 2
    images = [
        (p + pad, jnp.full_like(p, True, dtype=bool)),
        (pad - p, (p >= 1) & (p <= pad)),
        (2 * (H - 1) + pad - p, (p >= H - 1 - pad) & (p <= H - 2)),
    ]
    for q, valid in images:
        d = q - rp
        for tap in range(KSIZE):
            out = out + jnp.where(valid & (d == tap), k1d[tap],
                                  jnp.float32(0.0))
    return out * amp[..., None]


def kernel(_x_):
    xs = jax.nn.sigmoid(_x_)
    xs3 = xs.reshape(NPLANE, SUB, LANE)

    maxv, am = pl.pallas_call(
        _sample_body,
        grid=(GRID,),
        in_specs=[pl.BlockSpec((PB, SUB, LANE), lambda g: (g, 0, 0))],
        out_specs=[
            pl.BlockSpec((1, T, PB), lambda g: (g, 0, 0)),
            pl.BlockSpec((1, T, PB), lambda g: (g, 0, 0)),
        ],
        out_shape=[
            jax.ShapeDtypeStruct((GRID, T, PB), jnp.float32),
            jax.ShapeDtypeStruct((GRID, T, PB), jnp.int32),
        ],
        compiler_params=pltpu.CompilerParams(
            dimension_semantics=("parallel",)),
    )(xs3)

    maxv = maxv.transpose(1, 0, 2).reshape(T, B, C)
    am = am.transpose(1, 0, 2).reshape(T, B, C)

    i = (am // W).astype(jnp.float32) / H
    j = (am % W).astype(jnp.float32) / W
    y = jnp.stack([i, j], axis=3)
    vis_t = maxv >= 0.1
    y = y * vis_t[..., None].astype(jnp.float32)

    m_joints = jnp.mean(y, axis=0)
    v_joints = jnp.var(y, axis=0, ddof=1)
    vis = jnp.mean(vis_t.astype(jnp.float32), axis=0) > 0.5
    visf = vis[..., None].astype(jnp.float32)
    m_joints = m_joints * visf
    v_joints = v_joints * visf

    rows = jnp.clip(jnp.round(m_joints[..., 0] * H).astype(jnp.int32), 0, H - 1)
    cols = jnp.clip(jnp.round(m_joints[..., 1] * W).astype(jnp.int32), 0, W - 1)

    half = (KSIZE - 1) * 0.5
    xs1 = jnp.linspace(-half, half, KSIZE)
    pdf = jnp.exp(-0.5 * (xs1 / SIGMA) ** 2)
    k1d = (pdf / jnp.sum(pdf)).astype(jnp.float32)

    amp = vis.astype(jnp.float32)
    wr = _profiles(rows, amp, k1d)
    wc = _profiles(cols, amp, k1d)
    gmax = jnp.max(jnp.max(wr, axis=2) * jnp.max(wc, axis=2))
    wrn = (wr / gmax).reshape(NPLANE, H)
    wcf = wc.reshape(NPLANE, W)

    att = pl.pallas_call(
        _att_body,
        grid=(GRID,),
        in_specs=[
            pl.BlockSpec((PB, H), lambda g: (g, 0)),
            pl.BlockSpec((PB, W), lambda g: (g, 0)),
        ],
        out_specs=pl.BlockSpec((PB, H, W), lambda g: (g, 0, 0)),
        out_shape=jax.ShapeDtypeStruct((NPLANE, H, W), jnp.float32),
        compiler_params=pltpu.CompilerParams(
            dimension_semantics=("arbitrary",)),
    )(wrn, wcf)

    return att.reshape(B, C, H, W), m_joints, v_joints, vis


# speedup vs baseline: 1.1795x; 1.0566x over previous
import numpy as np

import jax
import jax.numpy as jnp
from jax.experimental import pallas as pl
from jax.experimental.pallas import tpu as pltpu

B, C, H, W = 4, 96, 224, 224
NPLANE = B * C
PIX = H * W
SUB, LANE = 392, 128
PB = 32
GRID = NPLANE // PB
T = 5
KSIZE = 7
SIGMA = 1.5


def _threefry_np(k1, k2, x0, x1):
    np.seterr(over="ignore")
    ks0 = np.uint32(k1)
    ks1 = np.uint32(k2)
    ks2 = ks0 ^ ks1 ^ np.uint32(0x1BD11BDA)

    def rot(x, r):
        return (x << np.uint32(r)) | (x >> np.uint32(32 - r))

    def rounds(a, b, rots):
        for r in rots:
            a = a + b
            b = rot(b, r)
            b = a ^ b
        return a, b

    R0 = (13, 15, 26, 6)
    R1 = (17, 29, 16, 24)
    x0 = x0 + ks0
    x1 = x1 + ks1
    x0, x1 = rounds(x0, x1, R0)
    x0 += ks1
    x1 += ks2 + np.uint32(1)
    x0, x1 = rounds(x0, x1, R1)
    x0 += ks2
    x1 += ks0 + np.uint32(2)
    x0, x1 = rounds(x0, x1, R0)
    x0 += ks0
    x1 += ks1 + np.uint32(3)
    x0, x1 = rounds(x0, x1, R1)
    x0 += ks1
    x1 += ks2 + np.uint32(4)
    x0, x1 = rounds(x0, x1, R0)
    x0 += ks2
    x1 += ks0 + np.uint32(5)
    return x0, x1


_KEYS = tuple(
    tuple(int(v) for v in _threefry_np(np.uint32(0), np.uint32(42),
                                       np.uint32(0), np.uint32(t)))
    for t in range(T)
)


def _rotl(x, r):
    return (x << jnp.uint32(r)) | (x >> jnp.uint32(32 - r))


def _tf_rounds(x0, x1, rots):
    for r in rots:
        x0 = x0 + x1
        x1 = _rotl(x1, r)
        x1 = x0 ^ x1
    return x0, x1


def _threefry_bits(k1, k2, cnt):
    R0 = (13, 15, 26, 6)
    R1 = (17, 29, 16, 24)
    ks0 = np.uint32(k1)
    ks1 = np.uint32(k2)
    ks2 = np.uint32(ks0 ^ ks1 ^ np.uint32(0x1BD11BDA))
    x0 = jnp.full_like(cnt, ks0)
    x1 = cnt + ks1
    x0, x1 = _tf_rounds(x0, x1, R0)
    x0 = x0 + ks1
    x1 = x1 + np.uint32(ks2 + np.uint32(1))
    x0, x1 = _tf_rounds(x0, x1, R1)
    x0 = x0 + ks2
    x1 = x1 + np.uint32(ks0 + np.uint32(2))
    x0, x1 = _tf_rounds(x0, x1, R0)
    x0 = x0 + ks0
    x1 = x1 + np.uint32(ks1 + np.uint32(3))
    x0, x1 = _tf_rounds(x0, x1, R1)
    x0 = x0 + ks1
    x1 = x1 + np.uint32(ks2 + np.uint32(4))
    x0, x1 = _tf_rounds(x0, x1, R0)
    x0 = x0 + ks2
    x1 = x1 + np.uint32(ks0 + np.uint32(5))
    return x0 ^ x1


def _sample_body(xs_ref, maxv_ref, am_ref):
    g = pl.program_id(0)
    xs = xs_ref[...]
    loc = (jax.lax.broadcasted_iota(jnp.int32, (PB, SUB, LANE), 1) * LANE
           + jax.lax.broadcasted_iota(jnp.int32, (PB, SUB, LANE), 2))
    plane = g * PB + jax.lax.broadcasted_iota(jnp.int32, (PB, SUB, LANE), 0)
    cnt = (plane * PIX + loc).astype(jnp.uint32)
    for t in range(T):
        k1, k2 = _KEYS[t]
        bits = _threefry_bits(k1, k2, cnt)
        fb = (bits >> jnp.uint32(9)) | jnp.uint32(0x3F800000)
        u = jax.lax.bitcast_convert_type(fb, jnp.float32) - jnp.float32(1.0)
        crnd = xs * u
        m = jnp.max(jnp.max(crnd, axis=2), axis=1)
        ismax = crnd == m[:, None, None]
        idx = jnp.where(ismax, loc, PIX)
        am = jnp.min(jnp.min(idx, axis=2), axis=1)
        maxv_ref[0, t, :] = m
        am_ref[0, t, :] = am


def _att_body(wr_ref, wc_ref, out_ref):
    out_ref[...] = wr_ref[...][:, :, None] * wc_ref[...][:, None, :]


def _profiles(pos, amp, k1d):
    rp = jnp.arange(H, dtype=jnp.int32)
    p = pos[..., None]
    out = jnp.zeros((B, C, H), jnp.float32)
    pad = KSIZE // 2
    images = [
        (p + pad, jnp.full_like(p, True, dtype=bool)),
        (pad - p, (p >= 1) & (p <= pad)),
        (2 * (H - 1) + pad - p, (p >= H - 1 - pad) & (p <= H - 2)),
    ]
    for q, valid in images:
        d = q - rp
        for tap in range(KSIZE):
            out = out + jnp.where(valid & (d == tap), k1d[tap],
                                  jnp.float32(0.0))
    return out * amp[..., None]


def kernel(_x_):
    xs = jax.nn.sigmoid(_x_)
    xs3 = xs.reshape(NPLANE, SUB, LANE)

    maxv, am = pl.pallas_call(
        _sample_body,
        grid=(GRID,),
        in_specs=[pl.BlockSpec((PB, SUB, LANE), lambda g: (g, 0, 0))],
        out_specs=[
            pl.BlockSpec((1, T, PB), lambda g: (g, 0, 0)),
            pl.BlockSpec((1, T, PB), lambda g: (g, 0, 0)),
        ],
        out_shape=[
            jax.ShapeDtypeStruct((GRID, T, PB), jnp.float32),
            jax.ShapeDtypeStruct((GRID, T, PB), jnp.int32),
        ],
        compiler_params=pltpu.CompilerParams(
            dimension_semantics=("parallel",)),
    )(xs3)

    maxv = maxv.transpose(1, 0, 2).reshape(T, B, C)
    am = am.transpose(1, 0, 2).reshape(T, B, C)

    i = (am // W).astype(jnp.float32) / H
    j = (am % W).astype(jnp.float32) / W
    y = jnp.stack([i, j], axis=3)
    vis_t = maxv >= 0.1
    y = y * vis_t[..., None].astype(jnp.float32)

    m_joints = jnp.mean(y, axis=0)
    v_joints = jnp.var(y, axis=0, ddof=1)
    vis = jnp.mean(vis_t.astype(jnp.float32), axis=0) > 0.5
    visf = vis[..., None].astype(jnp.float32)
    m_joints = m_joints * visf
    v_joints = v_joints * visf

    rows = jnp.clip(jnp.round(m_joints[..., 0] * H).astype(jnp.int32), 0, H - 1)
    cols = jnp.clip(jnp.round(m_joints[..., 1] * W).astype(jnp.int32), 0, W - 1)

    half = (KSIZE - 1) * 0.5
    xs1 = jnp.linspace(-half, half, KSIZE)
    pdf = jnp.exp(-0.5 * (xs1 / SIGMA) ** 2)
    k1d = (pdf / jnp.sum(pdf)).astype(jnp.float32)

    amp = vis.astype(jnp.float32)
    wr = _profiles(rows, amp, k1d)
    wc = _profiles(cols, amp, k1d)
    gmax = jnp.max(jnp.max(wr, axis=2) * jnp.max(wc, axis=2))
    wrn = (wr / gmax).reshape(NPLANE, H)
    wcf = wc.reshape(NPLANE, W)

    att = pl.pallas_call(
        _att_body,
        grid=(GRID,),
        in_specs=[
            pl.BlockSpec((PB, H), lambda g: (g, 0)),
            pl.BlockSpec((PB, W), lambda g: (g, 0)),
        ],
        out_specs=pl.BlockSpec((PB, H, W), lambda g: (g, 0, 0)),
        out_shape=jax.ShapeDtypeStruct((NPLANE, H, W), jnp.float32),
        compiler_params=pltpu.CompilerParams(
            dimension_semantics=("arbitrary",)),
    )(wrn, wcf)

    return att.reshape(B, C, H, W), m_joints, v_joints, vis


# speedup vs baseline: 1.2133x; 1.0287x over previous
import numpy as np

import jax
import jax.numpy as jnp
from jax.experimental import pallas as pl
from jax.experimental.pallas import tpu as pltpu

B, C, H, W = 4, 96, 224, 224
NPLANE = B * C
PIX = H * W
SUB, LANE = 392, 128
PB = 64
GRID = NPLANE // PB
T = 5
KSIZE = 7
SIGMA = 1.5


def _threefry_np(k1, k2, x0, x1):
    np.seterr(over="ignore")
    ks0 = np.uint32(k1)
    ks1 = np.uint32(k2)
    ks2 = ks0 ^ ks1 ^ np.uint32(0x1BD11BDA)

    def rot(x, r):
        return (x << np.uint32(r)) | (x >> np.uint32(32 - r))

    def rounds(a, b, rots):
        for r in rots:
            a = a + b
            b = rot(b, r)
            b = a ^ b
        return a, b

    R0 = (13, 15, 26, 6)
    R1 = (17, 29, 16, 24)
    x0 = x0 + ks0
    x1 = x1 + ks1
    x0, x1 = rounds(x0, x1, R0)
    x0 += ks1
    x1 += ks2 + np.uint32(1)
    x0, x1 = rounds(x0, x1, R1)
    x0 += ks2
    x1 += ks0 + np.uint32(2)
    x0, x1 = rounds(x0, x1, R0)
    x0 += ks0
    x1 += ks1 + np.uint32(3)
    x0, x1 = rounds(x0, x1, R1)
    x0 += ks1
    x1 += ks2 + np.uint32(4)
    x0, x1 = rounds(x0, x1, R0)
    x0 += ks2
    x1 += ks0 + np.uint32(5)
    return x0, x1


_KEYS = tuple(
    tuple(int(v) for v in _threefry_np(np.uint32(0), np.uint32(42),
                                       np.uint32(0), np.uint32(t)))
    for t in range(T)
)


def _rotl(x, r):
    return (x << jnp.uint32(r)) | (x >> jnp.uint32(32 - r))


def _tf_rounds(x0, x1, rots):
    for r in rots:
        x0 = x0 + x1
        x1 = _rotl(x1, r)
        x1 = x0 ^ x1
    return x0, x1


def _threefry_bits(k1, k2, cnt):
    R0 = (13, 15, 26, 6)
    R1 = (17, 29, 16, 24)
    ks0 = np.uint32(k1)
    ks1 = np.uint32(k2)
    ks2 = np.uint32(ks0 ^ ks1 ^ np.uint32(0x1BD11BDA))
    x0 = jnp.full_like(cnt, ks0)
    x1 = cnt + ks1
    x0, x1 = _tf_rounds(x0, x1, R0)
    x0 = x0 + ks1
    x1 = x1 + np.uint32(ks2 + np.uint32(1))
    x0, x1 = _tf_rounds(x0, x1, R1)
    x0 = x0 + ks2
    x1 = x1 + np.uint32(ks0 + np.uint32(2))
    x0, x1 = _tf_rounds(x0, x1, R0)
    x0 = x0 + ks0
    x1 = x1 + np.uint32(ks1 + np.uint32(3))
    x0, x1 = _tf_rounds(x0, x1, R1)
    x0 = x0 + ks1
    x1 = x1 + np.uint32(ks2 + np.uint32(4))
    x0, x1 = _tf_rounds(x0, x1, R0)
    x0 = x0 + ks2
    x1 = x1 + np.uint32(ks0 + np.uint32(5))
    return x0 ^ x1


def _sample_body(xs_ref, maxv_ref, am_ref):
    g = pl.program_id(0)
    xs = xs_ref[...]
    loc = (jax.lax.broadcasted_iota(jnp.int32, (PB, SUB, LANE), 1) * LANE
           + jax.lax.broadcasted_iota(jnp.int32, (PB, SUB, LANE), 2))
    plane = g * PB + jax.lax.broadcasted_iota(jnp.int32, (PB, SUB, LANE), 0)
    cnt = (plane * PIX + loc).astype(jnp.uint32)
    for t in range(T):
        k1, k2 = _KEYS[t]
        bits = _threefry_bits(k1, k2, cnt)
        fb = (bits >> jnp.uint32(9)) | jnp.uint32(0x3F800000)
        u = jax.lax.bitcast_convert_type(fb, jnp.float32) - jnp.float32(1.0)
        crnd = xs * u
        m = jnp.max(jnp.max(crnd, axis=2), axis=1)
        ismax = crnd == m[:, None, None]
        idx = jnp.where(ismax, loc, PIX)
        am = jnp.min(jnp.min(idx, axis=2), axis=1)
        maxv_ref[0, t, :] = m
        am_ref[0, t, :] = am


def _att_body(wr_ref, wc_ref, out_ref):
    out_ref[...] = wr_ref[...][:, :, None] * wc_ref[...][:, None, :]


def _profiles(pos, amp, k1d):
    rp = jnp.arange(H, dtype=jnp.int32)
    p = pos[..., None]
    out = jnp.zeros((B, C, H), jnp.float32)
    pad = KSIZE // 2
    images = [
        (p + pad, jnp.full_like(p, True, dtype=bool)),
        (pad - p, (p >= 1) & (p <= pad)),
        (2 * (H - 1) + pad - p, (p >= H - 1 - pad) & (p <= H - 2)),
    ]
    for q, valid in images:
        d = q - rp
        for tap in range(KSIZE):
            out = out + jnp.where(valid & (d == tap), k1d[tap],
                                  jnp.float32(0.0))
    return out * amp[..., None]


def kernel(_x_):
    xs = jax.nn.sigmoid(_x_)
    xs3 = xs.reshape(NPLANE, SUB, LANE)

    maxv, am = pl.pallas_call(
        _sample_body,
        grid=(GRID,),
        in_specs=[pl.BlockSpec((PB, SUB, LANE), lambda g: (g, 0, 0))],
        out_specs=[
            pl.BlockSpec((1, T, PB), lambda g: (g, 0, 0)),
            pl.BlockSpec((1, T, PB), lambda g: (g, 0, 0)),
        ],
        out_shape=[
            jax.ShapeDtypeStruct((GRID, T, PB), jnp.float32),
            jax.ShapeDtypeStruct((GRID, T, PB), jnp.int32),
        ],
        compiler_params=pltpu.CompilerParams(
            dimension_semantics=("parallel",)),
    )(xs3)

    maxv = maxv.transpose(1, 0, 2).reshape(T, B, C)
    am = am.transpose(1, 0, 2).reshape(T, B, C)

    i = (am // W).astype(jnp.float32) / H
    j = (am % W).astype(jnp.float32) / W
    y = jnp.stack([i, j], axis=3)
    vis_t = maxv >= 0.1
    y = y * vis_t[..., None].astype(jnp.float32)

    m_joints = jnp.mean(y, axis=0)
    v_joints = jnp.var(y, axis=0, ddof=1)
    vis = jnp.mean(vis_t.astype(jnp.float32), axis=0) > 0.5
    visf = vis[..., None].astype(jnp.float32)
    m_joints = m_joints * visf
    v_joints = v_joints * visf

    rows = jnp.clip(jnp.round(m_joints[..., 0] * H).astype(jnp.int32), 0, H - 1)
    cols = jnp.clip(jnp.round(m_joints[..., 1] * W).astype(jnp.int32), 0, W - 1)

    half = (KSIZE - 1) * 0.5
    xs1 = jnp.linspace(-half, half, KSIZE)
    pdf = jnp.exp(-0.5 * (xs1 / SIGMA) ** 2)
    k1d = (pdf / jnp.sum(pdf)).astype(jnp.float32)

    amp = vis.astype(jnp.float32)
    wr = _profiles(rows, amp, k1d)
    wc = _profiles(cols, amp, k1d)
    gmax = jnp.max(jnp.max(wr, axis=2) * jnp.max(wc, axis=2))
    wrn = (wr / gmax).reshape(NPLANE, H)
    wcf = wc.reshape(NPLANE, W)

    att = pl.pallas_call(
        _att_body,
        grid=(GRID,),
        in_specs=[
            pl.BlockSpec((PB, H), lambda g: (g, 0)),
            pl.BlockSpec((PB, W), lambda g: (g, 0)),
        ],
        out_specs=pl.BlockSpec((PB, H, W), lambda g: (g, 0, 0)),
        out_shape=jax.ShapeDtypeStruct((NPLANE, H, W), jnp.float32),
        compiler_params=pltpu.CompilerParams(
            dimension_semantics=("arbitrary",)),
    )(wrn, wcf)

    return att.reshape(B, C, H, W), m_joints, v_joints, vis
